# baseline (speedup 1.0000x reference)
import functools

import jax
import jax.numpy as jnp
from jax import lax
from jax.experimental import pallas as pl
from jax.experimental.pallas import tpu as pltpu
from jax.experimental.pallas import tpu_sc as plsc

_VOCAB = 1000000
_DIM = 64
_B = 16384
_L = 50

_NUM_CORES = 2
_NUM_SUBCORES = 16
_NW = _NUM_CORES * _NUM_SUBCORES
_BPW = _B // _NW
_TB = 128
_TPW = _BPW // _TB
_NBLK = _L * _TPW


def _make_gather_kernel():
  mesh = plsc.VectorSubcoreMesh(core_axis_name="c", subcore_axis_name="s")

  @functools.partial(
      pl.kernel,
      mesh=mesh,
      out_type=jax.ShapeDtypeStruct((_L, _DIM // 8, _B // 128, 8, 128),
                                    jnp.float32),
      scratch_types=[
          pltpu.VMEM((_L, _BPW), jnp.int32),
          pltpu.VMEM((_TB, _DIM), jnp.float32),
          pltpu.VMEM((_TB, _DIM), jnp.float32),
          pltpu.VMEM((_DIM // 8, 8, 128), jnp.float32),
          pltpu.VMEM((_DIM // 8, 8, 128), jnp.float32),
          pltpu.SemaphoreType.DMA,
          pltpu.SemaphoreType.DMA,
          pltpu.SemaphoreType.DMA,
          pltpu.SemaphoreType.DMA,
      ],
      compiler_params=pltpu.CompilerParams(use_tc_tiling_on_sc=False,
                                           needs_layout_passes=False),
  )
  def gather_kernel(idx_hbm, table_hbm, out_hbm, idx_v, rows0, rows1,
                    obuf0, obuf1, sem_g0, sem_g1, sem_o0, sem_o1):
    wid = lax.axis_index("s") * _NUM_CORES + lax.axis_index("c")
    base_b = wid * _BPW
    rows = (rows0, rows1)
    obuf = (obuf0, obuf1)
    sem_g = (sem_g0, sem_g1)
    sem_o = (sem_o0, sem_o1)
    iota = lax.iota(jnp.int32, 16)

    def block_lbt(j):
      return j // _TPW, j % _TPW

    def start_gather(j, b):
      l, bt = block_lbt(j)
      pltpu.async_copy(
          table_hbm.at[idx_v.at[l, pl.ds(bt * _TB, _TB)]], rows[b], sem_g[b])

    def wait_gather(j, b):
      l, bt = block_lbt(j)
      pltpu.make_async_copy(
          table_hbm.at[idx_v.at[l, pl.ds(bt * _TB, _TB)]], rows[b],
          sem_g[b]).wait()

    def start_out(j, b):
      l, bt = block_lbt(j)
      for dt in range(_DIM // 8):
        pltpu.async_copy(obuf[b].at[dt],
                         out_hbm.at[l, dt, wid * _TPW + bt], sem_o[b])

    def wait_out(j, b):
      l, bt = block_lbt(j)
      for dt in range(_DIM // 8):
        pltpu.make_async_copy(obuf[b].at[dt],
                              out_hbm.at[l, dt, wid * _TPW + bt],
                              sem_o[b]).wait()

    def transpose_block(b):
      r, o = rows[b], obuf[b]

      def body(cb, carry):
        rvec = iota + cb * 16
        for dt in range(_DIM // 8):
          for s in range(8):
            d = dt * 8 + s
            cvec = jnp.full((16,), d, jnp.int32)
            o[dt, s, pl.ds(cb * 16, 16)] = plsc.load_gather(r, [rvec, cvec])
        return carry

      lax.fori_loop(0, _TB // 16, body, 0)

    pltpu.sync_copy(idx_hbm.at[pl.ds(0, _L), pl.ds(base_b, _BPW)], idx_v)

    start_gather(0, 0)
    start_gather(1, 1)
    for j in (0, 1):
      b = j % 2
      wait_gather(j, b)
      transpose_block(b)
      start_gather(j + 2, b)
      start_out(j, b)

    def pair_body(g, carry):
      j = 2 * g + 2
      wait_gather(j, 0)
      wait_out(j - 2, 0)
      transpose_block(0)
      start_gather(j + 2, 0)
      start_out(j, 0)
      wait_gather(j + 1, 1)
      wait_out(j - 1, 1)
      transpose_block(1)
      start_gather(j + 3, 1)
      start_out(j + 1, 1)
      return carry

    lax.fori_loop(0, (_NBLK - 4) // 2, pair_body, 0)

    for j in (_NBLK - 2, _NBLK - 1):
      b = j % 2
      wait_gather(j, b)
      wait_out(j - 2, b)
      transpose_block(b)
      start_out(j, b)
    wait_out(_NBLK - 2, 0)
    wait_out(_NBLK - 1, 1)

  return gather_kernel


_gather = _make_gather_kernel()


@jax.jit
def kernel(input, table):
  idx_t = input.T.astype(jnp.int32)
  k = _gather(idx_t, table)
  return k.transpose(2, 4, 0, 1, 3).reshape(_B, _L, _DIM)


# speedup vs baseline: 1.4160x; 1.4160x over previous
import functools

import jax
import jax.numpy as jnp
from jax import lax
from jax.experimental import pallas as pl
from jax.experimental.pallas import tpu as pltpu
from jax.experimental.pallas import tpu_sc as plsc

_VOCAB = 1000000
_DIM = 64
_B = 16384
_L = 50

_NUM_CORES = 2
_NUM_SUBCORES = 16
_NW = _NUM_CORES * _NUM_SUBCORES
_BPW = _B // _NW
_TB = 128
_TPW = _BPW // _TB
_NBLK = _L * _TPW


def _make_gather_kernel():
  mesh = plsc.VectorSubcoreMesh(core_axis_name="c", subcore_axis_name="s")

  @functools.partial(
      pl.kernel,
      mesh=mesh,
      out_type=jax.ShapeDtypeStruct((_L, _DIM // 8, _B // 128, 8, 128),
                                    jnp.float32),
      scratch_types=[
          pltpu.VMEM((_L, _BPW), jnp.int32),
          pltpu.VMEM((_TB, _DIM), jnp.float32),
          pltpu.VMEM((_TB, _DIM), jnp.float32),
          pltpu.VMEM((_DIM, 128), jnp.float32),
          pltpu.VMEM((_DIM, 128), jnp.float32),
          pltpu.SemaphoreType.DMA,
          pltpu.SemaphoreType.DMA,
          pltpu.SemaphoreType.DMA,
          pltpu.SemaphoreType.DMA,
      ],
      compiler_params=pltpu.CompilerParams(use_tc_tiling_on_sc=False,
                                           needs_layout_passes=False),
  )
  def gather_kernel(idx_hbm, table_hbm, out_hbm, idx_v, rows0, rows1,
                    obuf0, obuf1, sem_g0, sem_g1, sem_o0, sem_o1):
    wid = lax.axis_index("s") * _NUM_CORES + lax.axis_index("c")
    base_b = wid * _BPW
    rows = (rows0, rows1)
    obuf = (obuf0, obuf1)
    sem_g = (sem_g0, sem_g1)
    sem_o = (sem_o0, sem_o1)
    iota = lax.iota(jnp.int32, 16)

    def block_lbt(j):
      return j // _TPW, j % _TPW

    def start_gather(j, b):
      l, bt = block_lbt(j)
      pltpu.async_copy(
          table_hbm.at[idx_v.at[l, pl.ds(bt * _TB, _TB)]], rows[b], sem_g[b])

    def wait_gather(j, b):
      l, bt = block_lbt(j)
      pltpu.make_async_copy(
          table_hbm.at[idx_v.at[l, pl.ds(bt * _TB, _TB)]], rows[b],
          sem_g[b]).wait()

    def start_out(j, b):
      l, bt = block_lbt(j)
      for dt in range(_DIM // 8):
        pltpu.async_copy(obuf[b].at[pl.ds(dt * 8, 8)],
                         out_hbm.at[l, dt, wid * _TPW + bt], sem_o[b])

    def wait_out(j, b):
      l, bt = block_lbt(j)
      for dt in range(_DIM // 8):
        pltpu.make_async_copy(obuf[b].at[pl.ds(dt * 8, 8)],
                              out_hbm.at[l, dt, wid * _TPW + bt],
                              sem_o[b]).wait()

    def transpose_block(b):
      r, o = rows[b], obuf[b]

      @plsc.parallel_loop(0, _DIM, unroll=8)
      def body(d):
        cvec = jnp.full((16,), 0, jnp.int32) + d
        for cb in range(_TB // 16):
          o[d, pl.ds(cb * 16, 16)] = plsc.load_gather(
              r, [iota + cb * 16, cvec])

    pltpu.sync_copy(idx_hbm.at[pl.ds(0, _L), pl.ds(base_b, _BPW)], idx_v)

    start_gather(0, 0)
    start_gather(1, 1)
    for j in (0, 1):
      b = j % 2
      wait_gather(j, b)
      transpose_block(b)
      start_gather(j + 2, b)
      start_out(j, b)

    def pair_body(g, carry):
      j = 2 * g + 2
      wait_gather(j, 0)
      wait_out(j - 2, 0)
      transpose_block(0)
      start_gather(j + 2, 0)
      start_out(j, 0)
      wait_gather(j + 1, 1)
      wait_out(j - 1, 1)
      transpose_block(1)
      start_gather(j + 3, 1)
      start_out(j + 1, 1)
      return carry

    lax.fori_loop(0, (_NBLK - 4) // 2, pair_body, 0)

    for j in (_NBLK - 2, _NBLK - 1):
      b = j % 2
      wait_gather(j, b)
      wait_out(j - 2, b)
      transpose_block(b)
      start_out(j, b)
    wait_out(_NBLK - 2, 0)
    wait_out(_NBLK - 1, 1)

  return gather_kernel


_gather = _make_gather_kernel()


@jax.jit
def kernel(input, table):
  idx_t = input.T.astype(jnp.int32)
  k = _gather(idx_t, table)
  return k.transpose(2, 4, 0, 1, 3).reshape(_B, _L, _DIM)


# speedup vs baseline: 1.5391x; 1.0869x over previous
import functools

import jax
import jax.numpy as jnp
from jax import lax
from jax.experimental import pallas as pl
from jax.experimental.pallas import tpu as pltpu
from jax.experimental.pallas import tpu_sc as plsc

_VOCAB = 1000000
_DIM = 64
_B = 16384
_L = 50

_NUM_CORES = 2
_NUM_SUBCORES = 16
_NW = _NUM_CORES * _NUM_SUBCORES
_BPW = _B // _NW
_NBUF = 3


def _make_gather_kernel():
  mesh = plsc.VectorSubcoreMesh(core_axis_name="c", subcore_axis_name="s")

  @functools.partial(
      pl.kernel,
      mesh=mesh,
      out_type=jax.ShapeDtypeStruct((_L, _B, _DIM), jnp.float32),
      scratch_types=[
          pltpu.VMEM((_L, _BPW), jnp.int32),
          pltpu.VMEM((_BPW, _DIM), jnp.float32),
          pltpu.VMEM((_BPW, _DIM), jnp.float32),
          pltpu.VMEM((_BPW, _DIM), jnp.float32),
          pltpu.SemaphoreType.DMA,
          pltpu.SemaphoreType.DMA,
          pltpu.SemaphoreType.DMA,
          pltpu.SemaphoreType.DMA,
          pltpu.SemaphoreType.DMA,
          pltpu.SemaphoreType.DMA,
      ],
      compiler_params=pltpu.CompilerParams(use_tc_tiling_on_sc=False),
  )
  def gather_kernel(idx_hbm, table_hbm, out_hbm, idx_v, rows0, rows1, rows2,
                    sem_g0, sem_g1, sem_g2, sem_o0, sem_o1, sem_o2):
    wid = lax.axis_index("s") * _NUM_CORES + lax.axis_index("c")
    base_b = wid * _BPW
    rows = (rows0, rows1, rows2)
    sem_g = (sem_g0, sem_g1, sem_g2)
    sem_o = (sem_o0, sem_o1, sem_o2)

    def start_gather(l, b):
      pltpu.async_copy(table_hbm.at[idx_v.at[l]], rows[b], sem_g[b])

    def wait_gather(l, b):
      pltpu.make_async_copy(table_hbm.at[idx_v.at[l]], rows[b],
                            sem_g[b]).wait()

    def start_out(l, b):
      pltpu.async_copy(rows[b], out_hbm.at[l, pl.ds(base_b, _BPW)], sem_o[b])

    def wait_out(l, b):
      pltpu.make_async_copy(rows[b], out_hbm.at[l, pl.ds(base_b, _BPW)],
                            sem_o[b]).wait()

    pltpu.sync_copy(idx_hbm.at[pl.ds(0, _L), pl.ds(base_b, _BPW)], idx_v)

    start_gather(0, 0)
    start_gather(1, 1)
    for l in range(_L):
      b = l % _NBUF
      if l + 2 < _L:
        if l >= 1:
          wait_out(l - 1, (l + 2) % _NBUF)
        start_gather(l + 2, (l + 2) % _NBUF)
      wait_gather(l, b)
      start_out(l, b)
    for l in (_L - 3, _L - 2, _L - 1):
      wait_out(l, l % _NBUF)

  return gather_kernel


_gather = _make_gather_kernel()


@jax.jit
def kernel(input, table):
  idx_t = input.T.astype(jnp.int32)
  k = _gather(idx_t, table)
  return k.transpose(1, 0, 2)
